# baseline (speedup 1.0000x reference)
import functools

import jax
import jax.numpy as jnp
from jax import lax
from jax.experimental import pallas as pl
from jax.experimental.pallas import tpu as pltpu
from jax.experimental.pallas import tpu_sc as plsc

_NW = 32
_SB = 128
_NJ = 200
_NBUF = 4


def _gather_t(bias, idx):
    V, D = bias.shape
    n_chunks = _NJ
    n_groups = n_chunks // _NBUF
    mesh = plsc.VectorSubcoreMesh(core_axis_name="c", subcore_axis_name="s")

    @functools.partial(
        pl.kernel,
        mesh=mesh,
        out_type=jax.ShapeDtypeStruct((_NJ, D // 8, _NW, 8, _SB), jnp.float32),
        scratch_types=(
            [pltpu.VMEM((_SB * _NJ,), jnp.int32)]
            + [pltpu.VMEM((_SB,), jnp.int32) for _ in range(_NBUF)]
            + [pltpu.VMEM((_SB, D), jnp.float32) for _ in range(_NBUF)]
            + [pltpu.VMEM((D // 8, 8, _SB), jnp.float32) for _ in range(_NBUF)]
            + [pltpu.SemaphoreType.DMA] * (2 * _NBUF)
        ),
        compiler_params=pltpu.CompilerParams(
            use_tc_tiling_on_sc=False, needs_layout_passes=False
        ),
    )
    def k(bias_hbm, idx_hbm, out_hbm, idx_v, *s):
        jidx = s[:_NBUF]
        rows = s[_NBUF:2 * _NBUF]
        tbuf = s[2 * _NBUF:3 * _NBUF]
        gsem = s[3 * _NBUF:4 * _NBUF]
        wsem = s[4 * _NBUF:5 * _NBUF]
        wid = lax.axis_index("s") * 2 + lax.axis_index("c")
        base = wid * (_SB * _NJ)
        pltpu.sync_copy(idx_hbm.at[pl.ds(base, _SB * _NJ)], idx_v)

        iota16 = lax.iota(jnp.int32, 16)
        s200 = iota16 * _NJ
        kvecs = [iota16 + kb * 16 for kb in range(_SB // 16)]

        def build_jidx(j, b):
            def kbody(kb, carry):
                pos = s200 + (kb * (16 * _NJ) + j)
                jidx[b][pl.ds(pl.multiple_of(kb * 16, 8), 16)] = plsc.load_gather(
                    idx_v, [pos]
                )
                return carry

            lax.fori_loop(0, _SB // 16, kbody, 0)

        def fire_gather(b):
            pltpu.make_async_copy(bias_hbm.at[jidx[b]], rows[b], gsem[b]).start()

        def wait_gather(b):
            pltpu.make_async_copy(bias_hbm.at[pl.ds(0, _SB)], rows[b], gsem[b]).wait()

        def transpose(b):
            def dbody(d, carry):
                col = jnp.full((16,), 0, jnp.int32) + d
                dblk = d // 8
                dsub = d % 8
                for kb in range(_SB // 16):
                    vals = plsc.load_gather(rows[b], [kvecs[kb], col])
                    tbuf[b][dblk, dsub, pl.ds(pl.multiple_of(kb * 16, 8), 16)] = vals
                return carry

            lax.fori_loop(0, D, dbody, 0)

        def fire_write(j, b):
            pltpu.make_async_copy(
                tbuf[b], out_hbm.at[j, :, wid], wsem[b]
            ).start()

        def wait_write(b):
            pltpu.make_async_copy(
                tbuf[b], out_hbm.at[0, :, 0], wsem[b]
            ).wait()

        for b in range(_NBUF):
            build_jidx(b, b)
            fire_gather(b)

        for b in range(_NBUF):
            wait_gather(b)
            transpose(b)
            fire_write(b, b)
            build_jidx(b + _NBUF, b)
            fire_gather(b)

        def body(g, carry):
            for b in range(_NBUF):
                j = g * _NBUF + b
                wait_gather(b)
                wait_write(b)
                transpose(b)
                fire_write(j, b)
                build_jidx(j + _NBUF, b)
                fire_gather(b)
            return carry

        lax.fori_loop(1, n_groups - 1, body, 0)

        for b in range(_NBUF):
            j = (n_groups - 1) * _NBUF + b
            wait_gather(b)
            wait_write(b)
            transpose(b)
            fire_write(j, b)
        for b in range(_NBUF):
            wait_write(b)

    return k(bias, idx)


def kernel(x, bias):
    idx = x.reshape(-1).astype(jnp.int32)
    v4 = _gather_t(bias, idx)
    return v4.transpose(2, 4, 0, 1, 3).reshape(x.shape + (bias.shape[1],))


# speedup vs baseline: 1.9788x; 1.9788x over previous
import functools

import jax
import jax.numpy as jnp
from jax import lax
from jax.experimental import pallas as pl
from jax.experimental.pallas import tpu as pltpu
from jax.experimental.pallas import tpu_sc as plsc

_NW = 32
_CHUNK = 128
_NBUF = 5


def _flat_gather(bias, idx):
    B = idx.shape[0]
    D = bias.shape[1]
    b_per_w = B // _NW
    n_chunks = b_per_w // _CHUNK
    n_groups = n_chunks // _NBUF
    mesh = plsc.VectorSubcoreMesh(core_axis_name="c", subcore_axis_name="s")

    @functools.partial(
        pl.kernel,
        mesh=mesh,
        out_type=jax.ShapeDtypeStruct((B, 2 * D), jnp.float32),
        scratch_types=(
            [pltpu.VMEM((b_per_w,), jnp.int32)]
            + [pltpu.VMEM((_CHUNK, D), jnp.float32) for _ in range(_NBUF)]
            + [pltpu.SemaphoreType.DMA] * (2 * _NBUF)
        ),
        compiler_params=pltpu.CompilerParams(use_tc_tiling_on_sc=False),
    )
    def k(bias_hbm, idx_hbm, out_hbm, idx_v, *s):
        rows = s[:_NBUF]
        gsem = s[_NBUF:2 * _NBUF]
        wsem = s[2 * _NBUF:3 * _NBUF]
        wid = lax.axis_index("s") * 2 + lax.axis_index("c")
        base = wid * b_per_w
        pltpu.sync_copy(idx_hbm.at[pl.ds(base, b_per_w)], idx_v)

        def fire_gather(j, b):
            pltpu.make_async_copy(
                bias_hbm.at[idx_v.at[pl.ds(j * _CHUNK, _CHUNK)]], rows[b], gsem[b]
            ).start()

        def wait_gather(b):
            pltpu.make_async_copy(bias_hbm.at[pl.ds(0, _CHUNK)], rows[b], gsem[b]).wait()

        def fire_write(j, b):
            pltpu.make_async_copy(
                rows[b],
                out_hbm.at[pl.ds(base + j * _CHUNK, _CHUNK), pl.ds(0, D)],
                wsem[b],
            ).start()

        def wait_write(b):
            pltpu.make_async_copy(
                rows[b], out_hbm.at[pl.ds(base, _CHUNK), pl.ds(0, D)], wsem[b]
            ).wait()

        for b in range(_NBUF):
            fire_gather(b, b)

        def outer(g, carry):
            for b in range(_NBUF):
                j = g * _NBUF + b
                wait_gather(b)
                fire_write(j, b)
                wait_write(b)
                fire_gather(j + _NBUF, b)
            return carry

        lax.fori_loop(0, n_groups - 1, outer, 0)
        for b in range(_NBUF):
            wait_gather(b)
            fire_write((n_groups - 1) * _NBUF + b, b)
            wait_write(b)

    return k(bias, idx)


def kernel(x, bias):
    idx = x.reshape(-1).astype(jnp.int32)
    outp = _flat_gather(bias, idx)
    return outp[:, : bias.shape[1]].reshape(x.shape + (bias.shape[1],))
